# baseline (speedup 1.0000x reference)
import functools

import jax
import jax.numpy as jnp
from jax import lax
from jax.experimental import pallas as pl
from jax.experimental.pallas import tpu as pltpu
from jax.experimental.pallas import tpu_sc as plsc

KNN_K = 32
NW = 32
CH = 128
NBUF = 3


def _sc_gather_call(featf, xyzf, idxf):
    P, C = featf.shape
    R = idxf.shape[0]
    PW = P // NW
    RW = R // NW
    NCH = RW // CH

    mesh = plsc.VectorSubcoreMesh(core_axis_name="c", subcore_axis_name="s")

    @functools.partial(
        pl.kernel,
        mesh=mesh,
        compiler_params=pltpu.CompilerParams(needs_layout_passes=False),
        out_type=(
            jax.ShapeDtypeStruct((R * 3,), jnp.float32),
            jax.ShapeDtypeStruct((R, C), jnp.float32),
        ),
        scratch_types=[
            pltpu.VMEM((RW,), jnp.int32),
            pltpu.VMEM((P * 3,), jnp.float32),
            pltpu.VMEM((NBUF, CH, C), jnp.float32),
            pltpu.VMEM((NBUF * CH * 3,), jnp.float32),
            pltpu.SemaphoreType.DMA,
            pltpu.SemaphoreType.DMA,
            pltpu.SemaphoreType.DMA,
            pltpu.SemaphoreType.DMA,
            pltpu.SemaphoreType.DMA,
            pltpu.SemaphoreType.DMA,
        ],
    )
    def k(featf_h, xyzf_h, idxf_h, nbw_h, nf_h, idx_w, xyz_all, fbuf, nbuf,
          sg0, sg1, sg2, so0, so1, so2):
        sg = [sg0, sg1, sg2]
        so = [so0, so1, so2]
        wid = lax.axis_index("s") * 2 + lax.axis_index("c")
        rbase = wid * RW
        pbase = wid * PW
        pltpu.sync_copy(idxf_h.at[pl.ds(rbase, RW)], idx_w)
        pltpu.sync_copy(xyzf_h, xyz_all)
        lane = lax.iota(jnp.int32, 16)

        def gstart(c, u):
            pltpu.async_copy(featf_h.at[idx_w.at[pl.ds(c * CH, CH)]],
                             fbuf.at[u], sg[u])

        def gwait(u):
            pltpu.make_async_copy(featf_h.at[pl.ds(0, CH)],
                                  fbuf.at[u], sg[u]).wait()

        def ostart(c, u):
            r0 = rbase + c * CH
            pltpu.async_copy(fbuf.at[u], nf_h.at[pl.ds(r0, CH)], so[u])
            pltpu.async_copy(nbuf.at[pl.ds(u * CH * 3, CH * 3)],
                             nbw_h.at[pl.ds(r0 * 3, CH * 3)], so[u])

        def owait(u):
            pltpu.make_async_copy(fbuf.at[u], nf_h.at[pl.ds(rbase, CH)],
                                  so[u]).wait()
            pltpu.make_async_copy(nbuf.at[pl.ds(u * CH * 3, CH * 3)],
                                  nbw_h.at[pl.ds(rbase * 3, CH * 3)],
                                  so[u]).wait()

        def compute_nbuf(c, u):
            for v in range(CH * 3 // 16):
                q = v * 16 + lane
                qd = q // 3
                coord = q - qd * 3
                rw = c * CH + qd
                nidx = plsc.load_gather(idx_w, [rw])
                g = plsc.load_gather(xyz_all, [nidx * 3 + coord])
                ctr = plsc.load_gather(
                    xyz_all, [(pbase + (rw >> 5)) * 3 + coord])
                nbuf[pl.ds(u * CH * 3 + v * 16, 16)] = g - ctr

        def process(c, u):
            compute_nbuf(c, u)
            gwait(u)
            ostart(c, u)
            un = (u + 1) % NBUF

            @pl.when(c + 1 < NCH)
            def _():
                @pl.when(c >= 2)
                def _():
                    owait(un)
                gstart(c + 1, un)

        gstart(0, 0)

        def trip(ct, carry):
            for u in range(NBUF):
                process(ct * NBUF + u, u)
            return carry

        nfull = (NCH // NBUF) * NBUF
        lax.fori_loop(0, NCH // NBUF, trip, 0)
        for c in range(nfull, NCH):
            process(jnp.int32(c), c % NBUF)
        for u in range(NBUF):
            owait(u)

    return k(featf, xyzf, idxf)


RB = 512
SEG = 128
SEGA = 32
CAND = 6


def _knn_body(xall_ref, xbt_ref, idx_ref):
    xall = xall_ref[0]
    xbt = xbt_ref[0]
    n = xall.shape[0]
    inner = jax.lax.dot_general(xall, xbt, (((1,), (0,)), ((), ())),
                                preferred_element_type=jnp.float32)
    sq_c = jnp.sum(xall * xall, axis=1, keepdims=True)
    sq_r = jnp.sum(xbt * xbt, axis=0, keepdims=True)
    d3 = (sq_c + sq_r - 2.0 * inner).reshape(SEG, SEGA, RB)
    a_id = jax.lax.broadcasted_iota(jnp.int32, (SEG, SEGA, RB), 1)
    sseg = jax.lax.broadcasted_iota(jnp.int32, (SEG, 1, RB), 0)
    big = jnp.int32(n)
    biga = jnp.int32(SEGA)
    inf = jnp.float32(jnp.inf)
    cvals, ccols = [], []
    for _ in range(CAND):
        m = jnp.min(d3, axis=1, keepdims=True)
        e = d3 == m
        t = jnp.where(e, a_id, biga)
        ja = jnp.min(t, axis=1, keepdims=True)
        cvals.append(m)
        ccols.append(sseg * SEGA + ja)
        d3 = jnp.where(e, inf, d3)
    cval = jnp.concatenate(cvals, axis=1)
    ccol = jnp.concatenate(ccols, axis=1)
    bofs = pl.program_id(0) * jnp.int32(n)
    rows = []
    for _ in range(KNN_K):
        m = jnp.min(cval, axis=(0, 1), keepdims=True)
        t = jnp.where(cval == m, ccol, big)
        j = jnp.min(t, axis=(0, 1), keepdims=True)
        rows.append(j[0] + bofs)
        cval = jnp.where(ccol == j, inf, cval)
    idx_ref[0] = jnp.concatenate(rows, axis=0)


def _knn_idx(xyz):
    B, N, _ = xyz.shape
    xyzp = jnp.pad(xyz, ((0, 0), (0, 0), (0, 5)))
    xyzpt = jnp.transpose(xyzp, (0, 2, 1))
    grid = (B, N // RB)
    idxt = pl.pallas_call(
        _knn_body,
        grid=grid,
        in_specs=[
            pl.BlockSpec((1, N, 8), lambda b, i: (b, 0, 0)),
            pl.BlockSpec((1, 8, RB), lambda b, i: (b, 0, i)),
        ],
        out_specs=pl.BlockSpec((1, KNN_K, RB), lambda b, i: (b, 0, i)),
        out_shape=jax.ShapeDtypeStruct((B, KNN_K, N), jnp.int32),
    )(xyzp, xyzpt)
    return jnp.transpose(idxt, (0, 2, 1)).reshape(B * N * KNN_K)


def kernel(xyz, feat):
    B, N, C = feat.shape
    P = B * N
    idxf = _knn_idx(xyz)
    featf = feat.reshape(P, C)
    nbw, nf = _sc_gather_call(featf, xyz.reshape(P * 3), idxf)
    neighborhood = nbw.reshape(B, N, KNN_K, 3)
    neighborhood_feat = nf.reshape(B, N, KNN_K, C)
    return neighborhood, neighborhood_feat


# speedup vs baseline: 1.1678x; 1.1678x over previous
import functools

import jax
import jax.numpy as jnp
from jax import lax
from jax.experimental import pallas as pl
from jax.experimental.pallas import tpu as pltpu
from jax.experimental.pallas import tpu_sc as plsc

KNN_K = 32
NW = 32
CH = 128
NBUF = 3


def _sc_gather_call(featf, xyzf, idxf):
    P, C = featf.shape
    R = idxf.shape[0]
    PW = P // NW
    RW = R // NW
    NCH = RW // CH

    mesh = plsc.VectorSubcoreMesh(core_axis_name="c", subcore_axis_name="s")

    @functools.partial(
        pl.kernel,
        mesh=mesh,
        compiler_params=pltpu.CompilerParams(needs_layout_passes=False),
        out_type=(
            jax.ShapeDtypeStruct((R * 3,), jnp.float32),
            jax.ShapeDtypeStruct((R, C), jnp.float32),
        ),
        scratch_types=[
            pltpu.VMEM((RW,), jnp.int32),
            pltpu.VMEM((P * 3,), jnp.float32),
            pltpu.VMEM((NBUF, CH, C), jnp.float32),
            pltpu.VMEM((NBUF * CH * 3,), jnp.float32),
            pltpu.SemaphoreType.DMA,
            pltpu.SemaphoreType.DMA,
            pltpu.SemaphoreType.DMA,
            pltpu.SemaphoreType.DMA,
            pltpu.SemaphoreType.DMA,
            pltpu.SemaphoreType.DMA,
        ],
    )
    def k(featf_h, xyzf_h, idxf_h, nbw_h, nf_h, idx_w, xyz_all, fbuf, nbuf,
          sg0, sg1, sg2, so0, so1, so2):
        sg = [sg0, sg1, sg2]
        so = [so0, so1, so2]
        wid = lax.axis_index("s") * 2 + lax.axis_index("c")
        rbase = wid * RW
        pbase = wid * PW
        pltpu.sync_copy(idxf_h.at[pl.ds(rbase, RW)], idx_w)
        pltpu.sync_copy(xyzf_h, xyz_all)
        lane = lax.iota(jnp.int32, 16)

        def gstart(c, u):
            pltpu.async_copy(featf_h.at[idx_w.at[pl.ds(c * CH, CH)]],
                             fbuf.at[u], sg[u])

        def gwait(u):
            pltpu.make_async_copy(featf_h.at[pl.ds(0, CH)],
                                  fbuf.at[u], sg[u]).wait()

        def ostart(c, u):
            r0 = rbase + c * CH
            pltpu.async_copy(fbuf.at[u], nf_h.at[pl.ds(r0, CH)], so[u])
            pltpu.async_copy(nbuf.at[pl.ds(u * CH * 3, CH * 3)],
                             nbw_h.at[pl.ds(r0 * 3, CH * 3)], so[u])

        def owait(u):
            pltpu.make_async_copy(fbuf.at[u], nf_h.at[pl.ds(rbase, CH)],
                                  so[u]).wait()
            pltpu.make_async_copy(nbuf.at[pl.ds(u * CH * 3, CH * 3)],
                                  nbw_h.at[pl.ds(rbase * 3, CH * 3)],
                                  so[u]).wait()

        def compute_nbuf(c, u):
            for v in range(CH * 3 // 16):
                q = v * 16 + lane
                qd = q // 3
                coord = q - qd * 3
                rw = c * CH + qd
                nidx = plsc.load_gather(idx_w, [rw])
                g = plsc.load_gather(xyz_all, [nidx * 3 + coord])
                ctr = plsc.load_gather(
                    xyz_all, [(pbase + (rw >> 5)) * 3 + coord])
                nbuf[pl.ds(u * CH * 3 + v * 16, 16)] = g - ctr

        def process(c, u):
            compute_nbuf(c, u)
            gwait(u)
            ostart(c, u)
            un = (u + 1) % NBUF

            @pl.when(c + 1 < NCH)
            def _():
                @pl.when(c >= 2)
                def _():
                    owait(un)
                gstart(c + 1, un)

        gstart(0, 0)

        def trip(ct, carry):
            for u in range(NBUF):
                process(ct * NBUF + u, u)
            return carry

        nfull = (NCH // NBUF) * NBUF
        lax.fori_loop(0, NCH // NBUF, trip, 0)
        for c in range(nfull, NCH):
            process(jnp.int32(c), c % NBUF)
        for u in range(NBUF):
            owait(u)

    return k(featf, xyzf, idxf)


RB = 256
SEG = 128
SEGA = 32
CAND = 6


def _knn_body(xall_ref, xbt_ref, idx_ref):
    xall = xall_ref[0]
    xbt = xbt_ref[0]
    n = xall.shape[0]
    inner = jax.lax.dot_general(xall, xbt, (((1,), (0,)), ((), ())),
                                preferred_element_type=jnp.float32)
    sq_c = jnp.sum(xall * xall, axis=1, keepdims=True)
    sq_r = jnp.sum(xbt * xbt, axis=0, keepdims=True)
    d3 = (sq_c + sq_r - 2.0 * inner).reshape(SEG, SEGA, RB)
    a_id = jax.lax.broadcasted_iota(jnp.int32, (SEG, SEGA, RB), 1)
    sseg = jax.lax.broadcasted_iota(jnp.int32, (SEG, 1, RB), 0)
    big = jnp.int32(n)
    biga = jnp.int32(SEGA)
    inf = jnp.float32(jnp.inf)
    cvals, ccols = [], []
    for _ in range(CAND):
        m = jnp.min(d3, axis=1, keepdims=True)
        e = d3 == m
        t = jnp.where(e, a_id, biga)
        ja = jnp.min(t, axis=1, keepdims=True)
        cvals.append(m)
        ccols.append(sseg * SEGA + ja)
        d3 = jnp.where(e, inf, d3)
    cval = jnp.concatenate(cvals, axis=1)
    ccol = jnp.concatenate(ccols, axis=1)
    bofs = pl.program_id(0) * jnp.int32(n)
    rows = []
    for _ in range(KNN_K):
        m = jnp.min(cval, axis=(0, 1), keepdims=True)
        t = jnp.where(cval == m, ccol, big)
        j = jnp.min(t, axis=(0, 1), keepdims=True)
        rows.append(j[0] + bofs)
        cval = jnp.where(ccol == j, inf, cval)
    idx_ref[0] = jnp.concatenate(rows, axis=0)


def _knn_idx(xyz):
    B, N, _ = xyz.shape
    xyzp = jnp.pad(xyz, ((0, 0), (0, 0), (0, 5)))
    xyzpt = jnp.transpose(xyzp, (0, 2, 1))
    grid = (B, N // RB)
    idxt = pl.pallas_call(
        _knn_body,
        grid=grid,
        in_specs=[
            pl.BlockSpec((1, N, 8), lambda b, i: (b, 0, 0)),
            pl.BlockSpec((1, 8, RB), lambda b, i: (b, 0, i)),
        ],
        out_specs=pl.BlockSpec((1, KNN_K, RB), lambda b, i: (b, 0, i)),
        out_shape=jax.ShapeDtypeStruct((B, KNN_K, N), jnp.int32),
    )(xyzp, xyzpt)
    return jnp.transpose(idxt, (0, 2, 1)).reshape(B * N * KNN_K)


def kernel(xyz, feat):
    B, N, C = feat.shape
    P = B * N
    idxf = _knn_idx(xyz)
    featf = feat.reshape(P, C)
    nbw, nf = _sc_gather_call(featf, xyz.reshape(P * 3), idxf)
    neighborhood = nbw.reshape(B, N, KNN_K, 3)
    neighborhood_feat = nf.reshape(B, N, KNN_K, C)
    return neighborhood, neighborhood_feat
